# baseline (speedup 1.0000x reference)
import functools

import jax
import jax.numpy as jnp
from jax import lax
from jax.experimental import pallas as pl
from jax.experimental.pallas import tpu as pltpu
from jax.experimental.pallas import tpu_sc as plsc

NC = 2
NS = 16
NW = NC * NS
LANES = 16
NBUF = 8


def kernel(inputs, table, bias):
    B, L = inputs.shape
    V, D = table.shape
    PAIR = 2
    GW = PAIR * L
    GWP = GW + (-GW % 8)
    n_groups = B // PAIR
    g_per_w = n_groups // NW
    bags_per_w = B // NW
    n_col = D // LANES

    idx = inputs.astype(jnp.int32).reshape(n_groups, GW)
    idx = jnp.pad(idx, ((0, 0), (0, GWP - GW)))

    mesh = plsc.VectorSubcoreMesh(core_axis_name="c", subcore_axis_name="s")

    @functools.partial(
        pl.kernel,
        out_type=jax.ShapeDtypeStruct((B, D), jnp.float32),
        mesh=mesh,
        compiler_params=pltpu.CompilerParams(use_tc_tiling_on_sc=False),
        scratch_types=[
            pltpu.VMEM((g_per_w, GWP), jnp.int32),
            pltpu.VMEM((NBUF, GWP, D), jnp.float32),
            pltpu.VMEM((bags_per_w, D), jnp.float32),
            pltpu.VMEM((D,), jnp.float32),
        ] + [pltpu.SemaphoreType.DMA] * NBUF,
    )
    def bow(table_hbm, idx_hbm, bias_hbm, out_hbm,
            idx_v, rows_v, out_v, bias_v, *sems):
        wid = lax.axis_index("s") * NC + lax.axis_index("c")
        gbase = wid * g_per_w

        pltpu.sync_copy(bias_hbm, bias_v)
        pltpu.sync_copy(idx_hbm.at[pl.ds(gbase, g_per_w)], idx_v)

        def start(s, b):
            pltpu.async_copy(table_hbm.at[idx_v.at[s]], rows_v.at[b], sems[b])

        def wait(b):
            pltpu.make_async_copy(
                table_hbm.at[idx_v.at[0]], rows_v.at[b], sems[b]).wait()

        def reduce_step(s, b):
            for p in range(PAIR):
                for c in range(n_col):
                    acc = bias_v[pl.ds(c * LANES, LANES)]
                    for l in range(L):
                        acc = acc + rows_v[b, p * L + l, pl.ds(c * LANES, LANES)]
                    out_v[s * PAIR + p, pl.ds(c * LANES, LANES)] = acc

        for b in range(NBUF):
            start(b, b)

        def body(g, carry):
            for b in range(NBUF):
                s = g * NBUF + b
                wait(b)
                reduce_step(s, b)
                s2 = s + NBUF

                @pl.when(s2 < g_per_w)
                def _():
                    start(s2, b)
            return carry

        lax.fori_loop(0, g_per_w // NBUF, body, None)
        pltpu.sync_copy(out_v, out_hbm.at[pl.ds(wid * bags_per_w, bags_per_w)])

    return bow(table, idx, bias)


# speedup vs baseline: 1.3468x; 1.3468x over previous
import functools

import jax
import jax.numpy as jnp
from jax import lax
from jax.experimental import pallas as pl
from jax.experimental.pallas import tpu as pltpu
from jax.experimental.pallas import tpu_sc as plsc

NC = 2
NS = 16
NW = NC * NS
LANES = 16
NBUF = 8


def kernel(inputs, table, bias):
    B, L = inputs.shape
    V, D = table.shape
    PAIR = 2
    GW = PAIR * L
    GWP = GW + (-GW % 8)
    n_groups = B // PAIR
    g_per_w = n_groups // NW
    bags_per_w = B // NW
    n_col = D // LANES

    idx = inputs.astype(jnp.int32).reshape(n_groups, GW)
    npad = GWP - GW
    pad = (jnp.arange(n_groups, dtype=jnp.int32)[:, None] * npad
           + jnp.arange(npad, dtype=jnp.int32)[None, :]) % V
    idx = jnp.concatenate([idx, pad], axis=1)

    mesh = plsc.VectorSubcoreMesh(core_axis_name="c", subcore_axis_name="s")

    @functools.partial(
        pl.kernel,
        out_type=jax.ShapeDtypeStruct((B, D), jnp.float32),
        mesh=mesh,
        compiler_params=pltpu.CompilerParams(use_tc_tiling_on_sc=False),
        scratch_types=[
            pltpu.VMEM((g_per_w, GWP), jnp.int32),
            pltpu.VMEM((NBUF, GWP, D), jnp.float32),
            pltpu.VMEM((bags_per_w, D), jnp.float32),
            pltpu.VMEM((D,), jnp.float32),
        ] + [pltpu.SemaphoreType.DMA] * NBUF,
    )
    def bow(table_hbm, idx_hbm, bias_hbm, out_hbm,
            idx_v, rows_v, out_v, bias_v, *sems):
        wid = lax.axis_index("s") * NC + lax.axis_index("c")
        gbase = wid * g_per_w

        pltpu.sync_copy(bias_hbm, bias_v)
        pltpu.sync_copy(idx_hbm.at[pl.ds(gbase, g_per_w)], idx_v)

        def start(s, b):
            pltpu.async_copy(table_hbm.at[idx_v.at[s]], rows_v.at[b], sems[b])

        def wait(b):
            pltpu.make_async_copy(
                table_hbm.at[idx_v.at[0]], rows_v.at[b], sems[b]).wait()

        def reduce_step(s, b):
            for p in range(PAIR):
                for c in range(n_col):
                    acc = bias_v[pl.ds(c * LANES, LANES)]
                    for l in range(L):
                        acc = acc + rows_v[b, p * L + l, pl.ds(c * LANES, LANES)]
                    out_v[s * PAIR + p, pl.ds(c * LANES, LANES)] = acc

        for b in range(NBUF):
            start(b, b)

        def body(g, carry):
            for b in range(NBUF):
                s = g * NBUF + b
                wait(b)
                reduce_step(s, b)
                s2 = s + NBUF

                @pl.when(s2 < g_per_w)
                def _():
                    start(s2, b)
            return carry

        lax.fori_loop(0, g_per_w // NBUF, body, None)
        pltpu.sync_copy(out_v, out_hbm.at[pl.ds(wid * bags_per_w, bags_per_w)])

    return bow(table, idx, bias)


# speedup vs baseline: 1.5166x; 1.1261x over previous
import functools

import jax
import jax.numpy as jnp
from jax import lax
from jax.experimental import pallas as pl
from jax.experimental.pallas import tpu as pltpu
from jax.experimental.pallas import tpu_sc as plsc

NC = 2
NS = 16
NW = NC * NS
LANES = 16
NBUF = 4


def kernel(inputs, table, bias):
    B, L = inputs.shape
    V, D = table.shape
    PAIR = 2
    GW = PAIR * L
    GWP = GW + (-GW % 8)
    n_groups = B // PAIR
    g_per_w = n_groups // NW
    bags_per_w = B // NW
    n_col = D // LANES

    table_w = jnp.pad(table, ((0, 0), (0, 128 - D)))

    idx = inputs.astype(jnp.int32).reshape(n_groups, GW)
    npad = GWP - GW
    pad = (jnp.arange(n_groups, dtype=jnp.int32)[:, None] * npad
           + jnp.arange(npad, dtype=jnp.int32)[None, :]) % V
    idx = jnp.concatenate([idx, pad], axis=1).reshape(-1)

    mesh = plsc.VectorSubcoreMesh(core_axis_name="c", subcore_axis_name="s")

    @functools.partial(
        pl.kernel,
        out_type=jax.ShapeDtypeStruct((B * D,), jnp.float32),
        mesh=mesh,
        compiler_params=pltpu.CompilerParams(use_tc_tiling_on_sc=True),
        scratch_types=[
            pltpu.VMEM((g_per_w * GWP,), jnp.int32),
            pltpu.VMEM((NBUF, GWP, 128), jnp.float32),
            pltpu.VMEM((bags_per_w * D,), jnp.float32),
            pltpu.VMEM((D,), jnp.float32),
        ] + [pltpu.SemaphoreType.DMA] * NBUF,
    )
    def bow(table_hbm, idx_hbm, bias_hbm, out_hbm,
            idx_v, rows_v, out_v, bias_v, *sems):
        wid = lax.axis_index("s") * NC + lax.axis_index("c")
        gbase = wid * g_per_w

        pltpu.sync_copy(bias_hbm, bias_v)
        pltpu.sync_copy(idx_hbm.at[pl.ds(gbase * GWP, g_per_w * GWP)], idx_v)

        def start(s, b):
            pltpu.async_copy(
                table_hbm.at[idx_v.at[pl.ds(s * GWP, GWP)]], rows_v.at[b],
                sems[b])

        def wait(b):
            pltpu.make_async_copy(
                table_hbm.at[idx_v.at[pl.ds(0, GWP)]], rows_v.at[b],
                sems[b]).wait()

        def reduce_step(s, b):
            for p in range(PAIR):
                for c in range(n_col):
                    acc = bias_v[pl.ds(c * LANES, LANES)]
                    for l in range(L):
                        acc = acc + rows_v[b, p * L + l, pl.ds(c * LANES, LANES)]
                    out_v[pl.ds((s * PAIR + p) * D + c * LANES, LANES)] = acc

        for b in range(NBUF):
            start(b, b)

        def body(g, carry):
            for b in range(NBUF):
                s = g * NBUF + b
                wait(b)
                reduce_step(s, b)
                s2 = s + NBUF

                @pl.when(s2 < g_per_w)
                def _():
                    start(s2, b)
            return carry

        lax.fori_loop(0, g_per_w // NBUF, body, None)
        pltpu.sync_copy(
            out_v, out_hbm.at[pl.ds(wid * bags_per_w * D, bags_per_w * D)])

    return bow(table_w, idx, bias).reshape(B, D)
